# baseline (speedup 1.0000x reference)
import functools

import jax
import jax.numpy as jnp
from jax import lax
from jax.experimental import pallas as pl
from jax.experimental.pallas import tpu as pltpu
from jax.experimental.pallas import tpu_sc as plsc

_SL = 8
_LN = 128


@functools.lru_cache(maxsize=None)
def _make_gather(B: int, H: int, D: int):
    info = plsc.get_sparse_core_info()
    nc, ns = info.num_cores, info.num_subcores
    nw = nc * ns
    HT, BT, TC = H // _SL, B // _LN, D // _SL
    tiles = HT * BT
    assert tiles % nw == 0
    per_w = tiles // nw
    mesh = plsc.VectorSubcoreMesh(core_axis_name="c", subcore_axis_name="s")

    @functools.partial(
        pl.kernel,
        mesh=mesh,
        out_type=jax.ShapeDtypeStruct((H, TC, BT, _SL, _LN), jnp.float32),
        scratch_types=[
            pltpu.VMEM((_SL, _LN), jnp.int32),
            pltpu.VMEM((_LN, D), jnp.float32),
            pltpu.VMEM((_LN, D), jnp.float32),
            pltpu.VMEM((D, _LN), jnp.float32),
            pltpu.VMEM((D, _LN), jnp.float32),
            pltpu.SemaphoreType.DMA,
            pltpu.SemaphoreType.DMA,
            pltpu.SemaphoreType.DMA,
            pltpu.SemaphoreType.DMA,
        ],
        compiler_params=pltpu.CompilerParams(
            use_tc_tiling_on_sc=False, needs_layout_passes=False),
    )
    def gather(xv_hbm, table_hbm, o_hbm, idx_t, rows0, rows1, tr0, tr1,
               g0, g1, w0, w1):
        wid = lax.axis_index("s") * nc + lax.axis_index("c")
        rows = (rows0, rows1)
        tr = (tr0, tr1)
        gs = (g0, g1)
        ws = (w0, w1)
        iotas = [lax.iota(jnp.int32, 16) + bg * 16 for bg in range(_LN // 16)]

        def transpose(src, dst):
            for c in range(D):
                cc = jnp.full((16,), c, jnp.int32)
                for bg in range(_LN // 16):
                    dst[c, pl.ds(bg * 16, 16)] = plsc.load_gather(
                        src, [iotas[bg], cc])

        def drain_writes(p):
            for tc in range(TC):
                pltpu.make_async_copy(
                    tr[p].at[pl.ds(tc * _SL, _SL)], o_hbm.at[0, tc, 0],
                    ws[p]).wait()

        def tile_body(i, carry):
            tid = wid * per_w + i
            ht = tid // BT
            bt = tid % BT
            pltpu.sync_copy(xv_hbm.at[ht, bt], idx_t)
            pltpu.async_copy(table_hbm.at[idx_t.at[0]], rows[0], gs[0])

            def hs_body(g2, carry2):
                for b in (0, 1):
                    hs = 2 * g2 + b

                    @pl.when(hs + 1 < _SL)
                    def _():
                        pltpu.async_copy(
                            table_hbm.at[idx_t.at[hs + 1]], rows[1 - b],
                            gs[1 - b])
                    pltpu.make_async_copy(
                        table_hbm.at[idx_t.at[hs]], rows[b], gs[b]).wait()

                    @pl.when(i * _SL + hs >= 2)
                    def _():
                        drain_writes(b)
                    transpose(rows[b], tr[b])
                    h = ht * _SL + hs
                    for tc in range(TC):
                        pltpu.async_copy(
                            tr[b].at[pl.ds(tc * _SL, _SL)],
                            o_hbm.at[h, tc, bt], ws[b])
                return carry2

            lax.fori_loop(0, _SL // 2, hs_body, 0)
            return carry

        lax.fori_loop(0, per_w, tile_body, 0)
        drain_writes(0)
        drain_writes(1)

    return gather


def kernel(x, table):
    B, H = x.shape
    D = table.shape[1]
    HT, BT, TC = H // _SL, B // _LN, D // _SL
    xv = (x.astype(jnp.int32).T
          .reshape(HT, _SL, BT, _LN).transpose(0, 2, 1, 3))
    o = _make_gather(B, H, D)(xv, table)
    return o.transpose(2, 4, 0, 1, 3).reshape(B, H, D)


# speedup vs baseline: 1.7390x; 1.7390x over previous
import functools

import jax
import jax.numpy as jnp
from jax import lax
from jax.experimental import pallas as pl
from jax.experimental.pallas import tpu as pltpu
from jax.experimental.pallas import tpu_sc as plsc

_IPG = 128
_K = 8
_CHUNK = _K * _IPG
_PD = 128


@functools.lru_cache(maxsize=None)
def _make_gather(total: int, dim: int):
    info = plsc.get_sparse_core_info()
    nc, ns = info.num_cores, info.num_subcores
    nw = nc * ns
    assert total % (nw * _CHUNK) == 0
    nb = total // (nw * _CHUNK)
    assert nb % 2 == 1 and nb >= 3
    mesh = plsc.VectorSubcoreMesh(core_axis_name="c", subcore_axis_name="s")

    @functools.partial(
        pl.kernel,
        mesh=mesh,
        out_type=jax.ShapeDtypeStruct((nw, nb, _CHUNK, _PD), jnp.float32),
        scratch_types=[
            pltpu.VMEM((nb * _K, _IPG), jnp.int32),
            pltpu.VMEM((_CHUNK, dim), jnp.float32),
            pltpu.VMEM((_CHUNK, dim), jnp.float32),
            pltpu.SemaphoreType.DMA,
            pltpu.SemaphoreType.DMA,
            pltpu.SemaphoreType.DMA,
            pltpu.SemaphoreType.DMA,
        ],
        compiler_params=pltpu.CompilerParams(use_tc_tiling_on_sc=False),
    )
    def gather(idx_hbm, tp4_hbm, out_hbm, idx_v, rows0, rows1, g0, g1,
               w0, w1):
        wid = lax.axis_index("s") * nc + lax.axis_index("c")
        rows = (rows0, rows1)
        g_sem = (g0, g1)
        w_sem = (w0, w1)

        def fire(cur, buf, sem):
            for j in range(_K):
                pltpu.async_copy(
                    tp4_hbm.at[idx_v.at[cur * _K + j]],
                    buf.at[pl.ds(j * _IPG, _IPG)],
                    sem,
                )

        def out_slab(b):
            return out_hbm.at[wid, b].at[:, pl.ds(0, dim)]

        def drain_gather(p):
            pltpu.make_async_copy(out_slab(0), rows[p], g_sem[p]).wait()

        def drain_wb(p):
            pltpu.make_async_copy(rows[p], out_slab(0), w_sem[p]).wait()

        pltpu.sync_copy(idx_hbm.at[wid], idx_v)

        fire(0, rows[0], g_sem[0])

        def body(g, carry):
            for b in (0, 1):
                cur = 1 + 2 * g + b
                cb = 1 - b
                pb = b

                @pl.when(cur >= 2)
                def _():
                    drain_wb(cb)
                fire(cur, rows[cb], g_sem[cb])
                drain_gather(pb)
                pltpu.async_copy(rows[pb], out_slab(cur - 1), w_sem[pb])
            return carry

        lax.fori_loop(0, (nb - 1) // 2, body, 0)

        drain_gather(0)
        pltpu.async_copy(rows[0], out_slab(nb - 1), w_sem[0])
        drain_wb(1)
        drain_wb(0)

    return gather


def kernel(x, table):
    b, h = x.shape
    d = table.shape[1]
    total = b * h
    gather = _make_gather(total, d)
    info = plsc.get_sparse_core_info()
    nw = info.num_cores * info.num_subcores
    ratio = _PD // d
    idx = (x.astype(jnp.int32) * ratio).reshape(nw, total // (nw * _IPG),
                                                _IPG)
    tpad = jnp.pad(table, ((0, 0), (0, _PD - d)))
    tp4 = tpad.reshape(table.shape[0] * ratio, d)
    out = gather(idx, tp4)
    return out.reshape(b, h, _PD)[:, :, :d]
